# baseline (speedup 1.0000x reference)
import functools

import jax
import jax.numpy as jnp
from jax import lax
from jax.experimental import pallas as pl
from jax.experimental.pallas import tpu as pltpu
from jax.experimental.pallas import tpu_sc as plsc

ORDER = 16
SUBSAMPLING_FACTOR = 2
NUM_FEATURES = 64
SUB_ORDER = ORDER // SUBSAMPLING_FACTOR

NC = 2
NS = 16
NW = NC * NS
ROWS_PER_W = 2


CHUNK = 16384
NBUF = 6
DEPTH = 3


def _make_sc_copy(n_out_rows, row):
    mesh = plsc.VectorSubcoreMesh(core_axis_name="c", subcore_axis_name="s")
    chunks_per_row = row // CHUNK
    n_chunks = ROWS_PER_W * chunks_per_row

    @functools.partial(
        pl.kernel,
        mesh=mesh,
        out_type=jax.ShapeDtypeStruct((n_out_rows, row), jnp.float32),
        scratch_types=[pltpu.VMEM_SHARED((NS, NBUF, CHUNK), jnp.float32)]
        + [pltpu.SemaphoreType.DMA] * (2 * NBUF),
    )
    def k(x_hbm, out_hbm, sbuf, *sems):
        sin = sems[:NBUF]
        sout = sems[NBUF:]
        sid = lax.axis_index("s")
        buf = sbuf.at[sid]
        wid = sid * NC + lax.axis_index("c")

        def mk(t):
            orow = wid * ROWS_PER_W + t // chunks_per_row
            irow = (orow // SUB_ORDER) * ORDER + (orow % SUB_ORDER) * SUBSAMPLING_FACTOR
            off = (t % chunks_per_row) * CHUNK
            b = t % NBUF
            cin = pltpu.make_async_copy(
                x_hbm.at[pl.ds(irow, 1), pl.ds(off, CHUNK)],
                buf.at[pl.ds(b, 1), :],
                sin[b],
            )
            cout = pltpu.make_async_copy(
                buf.at[pl.ds(b, 1), :],
                out_hbm.at[pl.ds(orow, 1), pl.ds(off, CHUNK)],
                sout[b],
            )
            return cin, cout

        copies = [mk(t) for t in range(n_chunks)]
        out_waited = [False] * n_chunks
        for t in range(min(DEPTH, n_chunks)):
            copies[t][0].start()
        for t in range(n_chunks):
            copies[t][0].wait()
            copies[t][1].start()
            u = t + DEPTH
            if u < n_chunks:
                if u >= NBUF:
                    copies[u - NBUF][1].wait()
                    out_waited[u - NBUF] = True
                copies[u][0].start()
        for t in range(n_chunks):
            if not out_waited[t]:
                copies[t][1].wait()

    return k


def kernel(x):
    B, C, H, W = x.shape
    row = NUM_FEATURES * H * W
    xr = x.reshape(B * ORDER, row)
    out = _make_sc_copy(B * SUB_ORDER, row)(xr)
    return out.reshape(B, SUB_ORDER * NUM_FEATURES, H, W)


# speedup vs baseline: 1.0174x; 1.0174x over previous
import functools

import jax
import jax.numpy as jnp
from jax import lax
from jax.experimental import pallas as pl
from jax.experimental.pallas import tpu as pltpu
from jax.experimental.pallas import tpu_sc as plsc

ORDER = 16
SUBSAMPLING_FACTOR = 2
NUM_FEATURES = 64
SUB_ORDER = ORDER // SUBSAMPLING_FACTOR

NC = 2
NS = 16
NW = NC * NS

GR = 2048
RPT = 16
NBUF = 3
DEPTH = 2


def _make_sc_copy(n_in_mega, n_out_mega, row):
    mesh = plsc.VectorSubcoreMesh(core_axis_name="c", subcore_axis_name="s")
    rows_per_mega = row // GR
    out_rows = n_out_mega * rows_per_mega
    rows_per_w = out_rows // NW
    n_tr = rows_per_w // RPT
    megas_per_w = rows_per_w // rows_per_mega
    tr_per_mega = rows_per_mega // RPT

    @functools.partial(
        pl.kernel,
        mesh=mesh,
        out_type=jax.ShapeDtypeStruct((out_rows, GR), jnp.float32),
        scratch_types=[pltpu.VMEM((RPT, GR), jnp.float32) for _ in range(NBUF)]
        + [pltpu.SemaphoreType.DMA] * (2 * NBUF),
    )
    def k(x_hbm, out_hbm, *bufs_and_sems):
        bufs = bufs_and_sems[:NBUF]
        sin = bufs_and_sems[NBUF : 2 * NBUF]
        sout = bufs_and_sems[2 * NBUF :]
        wid = lax.axis_index("s") * NC + lax.axis_index("c")
        lanes = lax.iota(jnp.int32, 16)

        def mk(t):
            o_mega = wid * megas_per_w + t // tr_per_mega
            i_mega = (o_mega // SUB_ORDER) * ORDER + (
                o_mega % SUB_ORDER
            ) * SUBSAMPLING_FACTOR
            k_off = (t % tr_per_mega) * RPT
            base_in = i_mega * rows_per_mega + k_off
            base_out = o_mega * rows_per_mega + k_off
            b = t % NBUF
            cin = pltpu.make_async_copy(
                x_hbm.at[base_in + lanes], bufs[b], sin[b]
            )
            cout = pltpu.make_async_copy(
                bufs[b], out_hbm.at[base_out + lanes], sout[b]
            )
            return cin, cout

        copies = [mk(t) for t in range(n_tr)]
        out_waited = [False] * n_tr
        for t in range(min(DEPTH, n_tr)):
            copies[t][0].start()
        for t in range(n_tr):
            copies[t][0].wait()
            copies[t][1].start()
            u = t + DEPTH
            if u < n_tr:
                if u >= NBUF:
                    copies[u - NBUF][1].wait()
                    out_waited[u - NBUF] = True
                copies[u][0].start()
        for t in range(n_tr):
            if not out_waited[t]:
                copies[t][1].wait()

    return k


def kernel(x):
    B, C, H, W = x.shape
    row = NUM_FEATURES * H * W
    xg = x.reshape(B * ORDER * (row // GR), GR)
    out = _make_sc_copy(B * ORDER, B * SUB_ORDER, row)(xg)
    return out.reshape(B, SUB_ORDER * NUM_FEATURES, H, W)


# speedup vs baseline: 1.3548x; 1.3317x over previous
import jax
import jax.numpy as jnp
from jax.experimental import pallas as pl

ORDER = 16
SUBSAMPLING_FACTOR = 2
NUM_FEATURES = 64
SUB_ORDER = ORDER // SUBSAMPLING_FACTOR


def _copy_kernel(in_ref, out_ref):
    out_ref[...] = in_ref[...]


def kernel(x):
    B, C, H, W = x.shape
    return pl.pallas_call(
        _copy_kernel,
        grid=(B, SUB_ORDER),
        in_specs=[
            pl.BlockSpec(
                (1, NUM_FEATURES, H, W),
                lambda b, g: (b, g * SUBSAMPLING_FACTOR, 0, 0),
            )
        ],
        out_specs=pl.BlockSpec((1, NUM_FEATURES, H, W), lambda b, g: (b, g, 0, 0)),
        out_shape=jax.ShapeDtypeStruct((B, SUB_ORDER * NUM_FEATURES, H, W), x.dtype),
    )(x)


# speedup vs baseline: 1.3853x; 1.0225x over previous
import jax
import jax.numpy as jnp
from jax.experimental import pallas as pl
from jax.experimental.pallas import tpu as pltpu

ORDER = 16
SUBSAMPLING_FACTOR = 2
NUM_FEATURES = 64
SUB_ORDER = ORDER // SUBSAMPLING_FACTOR

NBUF = 8
DEPTH = 4


def _dma_kernel(x_hbm, out_hbm, buf, *sems):
    sin = sems[:NBUF]
    sout = sems[NBUF:]
    B = x_hbm.shape[0]
    n_chunks = B * SUB_ORDER

    def mk(t):
        b, g = divmod(t, SUB_ORDER)
        r = t % NBUF
        cin = pltpu.make_async_copy(
            x_hbm.at[pl.ds(b, 1), pl.ds(g * SUBSAMPLING_FACTOR * NUM_FEATURES, NUM_FEATURES)],
            buf.at[pl.ds(r, 1)],
            sin[r],
        )
        cout = pltpu.make_async_copy(
            buf.at[pl.ds(r, 1)],
            out_hbm.at[pl.ds(b, 1), pl.ds(g * NUM_FEATURES, NUM_FEATURES)],
            sout[r],
        )
        return cin, cout

    copies = [mk(t) for t in range(n_chunks)]
    out_waited = [False] * n_chunks
    for t in range(min(DEPTH, n_chunks)):
        copies[t][0].start()
    for t in range(n_chunks):
        copies[t][0].wait()
        copies[t][1].start()
        u = t + DEPTH
        if u < n_chunks:
            if u >= NBUF:
                copies[u - NBUF][1].wait()
                out_waited[u - NBUF] = True
            copies[u][0].start()
    for t in range(n_chunks):
        if not out_waited[t]:
            copies[t][1].wait()


def kernel(x):
    B, C, H, W = x.shape
    return pl.pallas_call(
        _dma_kernel,
        in_specs=[pl.BlockSpec(memory_space=pltpu.MemorySpace.HBM)],
        out_specs=pl.BlockSpec(memory_space=pltpu.MemorySpace.HBM),
        out_shape=jax.ShapeDtypeStruct((B, SUB_ORDER * NUM_FEATURES, H, W), x.dtype),
        scratch_shapes=[pltpu.VMEM((NBUF, NUM_FEATURES, H, W), jnp.float32)]
        + [pltpu.SemaphoreType.DMA] * (2 * NBUF),
    )(x)
